# baseline (speedup 1.0000x reference)
import functools

import jax
import jax.numpy as jnp
from jax import lax
from jax.experimental import pallas as pl
from jax.experimental.pallas import tpu as pltpu
from jax.experimental.pallas import tpu_sc as plsc

N_NODES = 100000
N_EDGES = 1600000
D_NODE = 128
D_EDGE = 16
N_GRAPHS = 64
NSEG = N_GRAPHS + 1
TRASH = N_GRAPHS

NW = 32

NB = 256
N_BLOCKS = 12
N_PER_W = N_BLOCKS * NB
N_EXTRA_BASE = NW * N_PER_W
N_TAIL_BASE = 99840
N_TAIL_ROWS = N_NODES - N_TAIL_BASE
N_PAD = 100096

EBLK = 32000
E_GRID = N_EDGES // EBLK


def _sc_body(x_hbm, nidx_hbm, zn_hbm, outn_hbm,
             ndata, nidx_b, acc_n, sem0, sem1):
    cid = lax.axis_index("c")
    sid = lax.axis_index("s")
    w = sid * 2 + cid
    sems = (sem0, sem1)

    @pl.when(sid == 0)
    def _():
        pltpu.sync_copy(zn_hbm, acc_n)

    plsc.subcore_barrier()

    def n_start(i, buf):
        base = pl.multiple_of(w * N_PER_W + i * NB, 128)
        pltpu.async_copy(x_hbm.at[pl.ds(base, NB)], ndata.at[buf], sems[buf])
        pltpu.async_copy(nidx_hbm.at[pl.ds(base, NB)], nidx_b.at[buf],
                         sems[buf])

    def n_wait(buf):
        pltpu.make_async_copy(x_hbm.at[pl.ds(0, NB)], ndata.at[buf],
                              sems[buf]).wait()
        pltpu.make_async_copy(nidx_hbm.at[pl.ds(0, NB)], nidx_b.at[buf],
                              sems[buf]).wait()

    def n_scat(buf):
        pltpu.sync_copy(ndata.at[buf], acc_n.at[nidx_b.at[buf]], add=True)

    n_start(0, 0)

    def n_pair(p, carry):
        n_start(2 * p + 1, 1)
        n_wait(0)
        n_scat(0)

        @pl.when(p < N_BLOCKS // 2 - 1)
        def _():
            n_start(2 * p + 2, 0)

        n_wait(1)
        n_scat(1)
        return carry

    lax.fori_loop(0, N_BLOCKS // 2, n_pair, 0)

    @pl.when(w == 0)
    def _():
        def body(k, carry):
            base = pl.multiple_of(N_EXTRA_BASE + k * NB, 128)
            pltpu.sync_copy(x_hbm.at[pl.ds(base, NB)], ndata.at[0])
            pltpu.sync_copy(nidx_hbm.at[pl.ds(base, NB)], nidx_b.at[0])
            n_scat(0)
            return carry

        lax.fori_loop(0, 4, body, 0)

    @pl.when(w == 1)
    def _():
        for base in (99328, 99584):
            pltpu.sync_copy(x_hbm.at[pl.ds(base, NB)], ndata.at[0])
            pltpu.sync_copy(nidx_hbm.at[pl.ds(base, NB)], nidx_b.at[0])
            n_scat(0)
        pltpu.sync_copy(x_hbm.at[pl.ds(N_TAIL_BASE, N_TAIL_ROWS)],
                        ndata.at[0, pl.ds(0, N_TAIL_ROWS)])
        pltpu.sync_copy(nidx_hbm.at[pl.ds(N_TAIL_BASE, NB)], nidx_b.at[0])
        n_scat(0)

    plsc.subcore_barrier()

    @pl.when(sid == 0)
    def _():
        pltpu.sync_copy(acc_n.at[pl.ds(0, N_GRAPHS)], outn_hbm.at[cid])


_sc_node_sums = functools.partial(
    pl.kernel,
    out_type=jax.ShapeDtypeStruct((2, N_GRAPHS, D_NODE), jnp.float32),
    mesh=plsc.VectorSubcoreMesh(core_axis_name="c", subcore_axis_name="s",
                                num_cores=2, num_subcores=16),
    compiler_params=pltpu.CompilerParams(use_tc_tiling_on_sc=False),
    scratch_types=[
        pltpu.VMEM((2, NB, D_NODE), jnp.float32),
        pltpu.VMEM((2, NB), jnp.int32),
        pltpu.VMEM_SHARED((NSEG, D_NODE), jnp.float32),
        pltpu.SemaphoreType.DMA,
        pltpu.SemaphoreType.DMA,
    ],
)(_sc_body)


def _edge_body(et_ref, lo_ref, hi_ref, out_ref):
    i = pl.program_id(0)

    @pl.when(i == 0)
    def _():
        out_ref[...] = jnp.zeros_like(out_ref)

    off = i * EBLK
    lo = jnp.clip(lo_ref[...] - off, 0, EBLK).astype(jnp.int16)
    hi = jnp.clip(hi_ref[...] - off, 0, EBLK).astype(jnp.int16)
    bi = jax.lax.broadcasted_iota(jnp.int16, (N_GRAPHS, EBLK), 1)
    onehot = jnp.where((bi >= lo) & (bi < hi), jnp.bfloat16(1),
                       jnp.bfloat16(0))
    out_ref[...] += jax.lax.dot_general(
        onehot, et_ref[...].astype(jnp.bfloat16), (((1,), (1,)), ((), ())),
        preferred_element_type=jnp.float32)


def _mlp_body(u_ref, pn_ref, pe_ref, w1_ref, b1_ref, w2_ref, b2_ref,
              g_ref, bt_ref, out_ref):
    nodes = pn_ref[0] + pn_ref[1]
    h = (jnp.dot(u_ref[...], w1_ref[0:32, :],
                 preferred_element_type=jnp.float32)
         + jnp.dot(nodes, w1_ref[32:160, :],
                   preferred_element_type=jnp.float32)
         + jnp.dot(pe_ref[...], w1_ref[160:176, :],
                   preferred_element_type=jnp.float32)
         + b1_ref[...])
    h = jnp.maximum(h, 0.0)
    h = jnp.dot(h, w2_ref[...], preferred_element_type=jnp.float32) + b2_ref[...]
    mean = jnp.mean(h, axis=-1, keepdims=True)
    var = jnp.mean((h - mean) ** 2, axis=-1, keepdims=True)
    out_ref[...] = ((h - mean) * jax.lax.rsqrt(var + 1e-5) * g_ref[...]
                    + bt_ref[...])


def kernel(x, edge_index, edge_attr, u, node_index, W1, b1, W2, b2,
           ln_gamma, ln_beta):
    nidx_pad = jnp.concatenate(
        [node_index, jnp.full((N_PAD - N_NODES,), TRASH, dtype=jnp.int32)])
    zn = jnp.zeros((NSEG, D_NODE), jnp.float32)

    pn = _sc_node_sums(x, nidx_pad, zn)

    et = edge_attr.T
    bounds = jnp.searchsorted(edge_index, jnp.arange(N_GRAPHS + 1),
                              method="scan_unrolled").astype(jnp.int32)
    lo = bounds[:N_GRAPHS].reshape(N_GRAPHS, 1)
    hi = bounds[1:].reshape(N_GRAPHS, 1)
    eagg = pl.pallas_call(
        _edge_body,
        grid=(E_GRID,),
        in_specs=[
            pl.BlockSpec((D_EDGE, EBLK), lambda i: (0, i)),
            pl.BlockSpec((N_GRAPHS, 1), lambda i: (0, 0)),
            pl.BlockSpec((N_GRAPHS, 1), lambda i: (0, 0)),
        ],
        out_specs=pl.BlockSpec((N_GRAPHS, D_EDGE), lambda i: (0, 0)),
        out_shape=jax.ShapeDtypeStruct((N_GRAPHS, D_EDGE), jnp.float32),
    )(et, lo, hi)

    out = pl.pallas_call(
        _mlp_body,
        out_shape=jax.ShapeDtypeStruct((N_GRAPHS, 128), jnp.float32),
    )(u, pn, eagg, W1, b1.reshape(1, -1), W2, b2.reshape(1, -1),
      ln_gamma.reshape(1, -1), ln_beta.reshape(1, -1))
    return out


# speedup vs baseline: 1.7481x; 1.7481x over previous
import functools

import jax
import jax.numpy as jnp
from jax import lax
from jax.experimental import pallas as pl
from jax.experimental.pallas import tpu as pltpu
from jax.experimental.pallas import tpu_sc as plsc

N_NODES = 100000
N_EDGES = 1600000
D_NODE = 128
D_EDGE = 16
N_GRAPHS = 64
NSEG = N_GRAPHS + 1
TRASH = N_GRAPHS

NW = 32

NB = 256
N_BLOCKS = 12
N_PER_W = N_BLOCKS * NB
N_EXTRA_BASE = NW * N_PER_W
N_TAIL_BASE = 99840
N_TAIL_ROWS = N_NODES - N_TAIL_BASE
N_PAD = 100096

EBLK = 32000
E_GRID = N_EDGES // EBLK


def _sc_body(x_hbm, nidx_hbm, zn_hbm, outn_hbm,
             ndata, nidx_b, acc_n, sem0, sem1):
    cid = lax.axis_index("c")
    sid = lax.axis_index("s")
    w = sid * 2 + cid
    sems = (sem0, sem1)

    @pl.when(sid == 0)
    def _():
        pltpu.sync_copy(zn_hbm, acc_n)

    plsc.subcore_barrier()

    def n_start(i, buf):
        base = pl.multiple_of(w * N_PER_W + i * NB, 128)
        pltpu.async_copy(x_hbm.at[pl.ds(base, NB)], ndata.at[buf], sems[buf])
        pltpu.async_copy(nidx_hbm.at[pl.ds(base, NB)], nidx_b.at[buf],
                         sems[buf])

    def n_wait(buf):
        pltpu.make_async_copy(x_hbm.at[pl.ds(0, NB)], ndata.at[buf],
                              sems[buf]).wait()
        pltpu.make_async_copy(nidx_hbm.at[pl.ds(0, NB)], nidx_b.at[buf],
                              sems[buf]).wait()

    def n_scat(buf):
        pltpu.sync_copy(ndata.at[buf], acc_n.at[nidx_b.at[buf]], add=True)

    n_start(0, 0)

    def n_pair(p, carry):
        n_start(2 * p + 1, 1)
        n_wait(0)
        n_scat(0)

        @pl.when(p < N_BLOCKS // 2 - 1)
        def _():
            n_start(2 * p + 2, 0)

        n_wait(1)
        n_scat(1)
        return carry

    lax.fori_loop(0, N_BLOCKS // 2, n_pair, 0)

    @pl.when(w == 0)
    def _():
        def body(k, carry):
            base = pl.multiple_of(N_EXTRA_BASE + k * NB, 128)
            pltpu.sync_copy(x_hbm.at[pl.ds(base, NB)], ndata.at[0])
            pltpu.sync_copy(nidx_hbm.at[pl.ds(base, NB)], nidx_b.at[0])
            n_scat(0)
            return carry

        lax.fori_loop(0, 4, body, 0)

    @pl.when(w == 1)
    def _():
        for base in (99328, 99584):
            pltpu.sync_copy(x_hbm.at[pl.ds(base, NB)], ndata.at[0])
            pltpu.sync_copy(nidx_hbm.at[pl.ds(base, NB)], nidx_b.at[0])
            n_scat(0)
        pltpu.sync_copy(x_hbm.at[pl.ds(N_TAIL_BASE, N_TAIL_ROWS)],
                        ndata.at[0, pl.ds(0, N_TAIL_ROWS)])
        pltpu.sync_copy(nidx_hbm.at[pl.ds(N_TAIL_BASE, NB)], nidx_b.at[0])
        n_scat(0)

    plsc.subcore_barrier()

    @pl.when(sid == 0)
    def _():
        pltpu.sync_copy(acc_n.at[pl.ds(0, N_GRAPHS)], outn_hbm.at[cid])


_sc_node_sums = functools.partial(
    pl.kernel,
    out_type=jax.ShapeDtypeStruct((2, N_GRAPHS, D_NODE), jnp.float32),
    mesh=plsc.VectorSubcoreMesh(core_axis_name="c", subcore_axis_name="s",
                                num_cores=2, num_subcores=16),
    compiler_params=pltpu.CompilerParams(use_tc_tiling_on_sc=False),
    scratch_types=[
        pltpu.VMEM((2, NB, D_NODE), jnp.float32),
        pltpu.VMEM((2, NB), jnp.int32),
        pltpu.VMEM_SHARED((NSEG, D_NODE), jnp.float32),
        pltpu.SemaphoreType.DMA,
        pltpu.SemaphoreType.DMA,
    ],
)(_sc_body)


ESUB = 4000


def _edge_body(et_ref, idx_ref, out_ref):
    i = pl.program_id(0)

    @pl.when(i == 0)
    def _():
        out_ref[...] = jnp.zeros_like(out_ref)

    ids = idx_ref[0].astype(jnp.int16)
    segs = jax.lax.broadcasted_iota(jnp.int16, (N_GRAPHS, ESUB), 0)
    etb = et_ref[...].astype(jnp.bfloat16)
    acc = jnp.zeros((N_GRAPHS, D_EDGE), jnp.float32)
    for r in range(EBLK // ESUB):
        onehot = jnp.where(ids[r:r + 1, :] == segs, jnp.bfloat16(1),
                           jnp.bfloat16(0))
        acc += jax.lax.dot_general(
            onehot, etb[:, r * ESUB:(r + 1) * ESUB],
            (((1,), (1,)), ((), ())), preferred_element_type=jnp.float32)
    out_ref[...] += acc


def _mlp_body(u_ref, pn_ref, pe_ref, w1_ref, b1_ref, w2_ref, b2_ref,
              g_ref, bt_ref, out_ref):
    nodes = pn_ref[0] + pn_ref[1]
    h = (jnp.dot(u_ref[...], w1_ref[0:32, :],
                 preferred_element_type=jnp.float32)
         + jnp.dot(nodes, w1_ref[32:160, :],
                   preferred_element_type=jnp.float32)
         + jnp.dot(pe_ref[...], w1_ref[160:176, :],
                   preferred_element_type=jnp.float32)
         + b1_ref[...])
    h = jnp.maximum(h, 0.0)
    h = jnp.dot(h, w2_ref[...], preferred_element_type=jnp.float32) + b2_ref[...]
    mean = jnp.mean(h, axis=-1, keepdims=True)
    var = jnp.mean((h - mean) ** 2, axis=-1, keepdims=True)
    out_ref[...] = ((h - mean) * jax.lax.rsqrt(var + 1e-5) * g_ref[...]
                    + bt_ref[...])


def kernel(x, edge_index, edge_attr, u, node_index, W1, b1, W2, b2,
           ln_gamma, ln_beta):
    nidx_pad = jnp.concatenate(
        [node_index, jnp.full((N_PAD - N_NODES,), TRASH, dtype=jnp.int32)])
    zn = jnp.zeros((NSEG, D_NODE), jnp.float32)

    pn = _sc_node_sums(x, nidx_pad, zn)

    et = edge_attr.T
    eidx3 = edge_index.reshape(E_GRID, EBLK // ESUB, ESUB)
    eagg = pl.pallas_call(
        _edge_body,
        grid=(E_GRID,),
        in_specs=[
            pl.BlockSpec((D_EDGE, EBLK), lambda i: (0, i)),
            pl.BlockSpec((1, EBLK // ESUB, ESUB), lambda i: (i, 0, 0)),
        ],
        out_specs=pl.BlockSpec((N_GRAPHS, D_EDGE), lambda i: (0, 0)),
        out_shape=jax.ShapeDtypeStruct((N_GRAPHS, D_EDGE), jnp.float32),
    )(et, eidx3)

    out = pl.pallas_call(
        _mlp_body,
        out_shape=jax.ShapeDtypeStruct((N_GRAPHS, 128), jnp.float32),
    )(u, pn, eagg, W1, b1.reshape(1, -1), W2, b2.reshape(1, -1),
      ln_gamma.reshape(1, -1), ln_beta.reshape(1, -1))
    return out


# speedup vs baseline: 1.9638x; 1.1234x over previous
import functools

import jax
import jax.numpy as jnp
from jax import lax
from jax.experimental import pallas as pl
from jax.experimental.pallas import tpu as pltpu
from jax.experimental.pallas import tpu_sc as plsc

N_NODES = 100000
N_EDGES = 1600000
D_NODE = 128
D_EDGE = 16
N_GRAPHS = 64
NSEG = N_GRAPHS + 1
TRASH = N_GRAPHS

NW = 32

NB = 256
N_BLOCKS = 12
N_PER_W = N_BLOCKS * NB
N_EXTRA_BASE = NW * N_PER_W
N_TAIL_BASE = 99840
N_TAIL_ROWS = N_NODES - N_TAIL_BASE
N_PAD = 100096

EBLK = 64000
E_GRID = N_EDGES // EBLK


def _sc_body(x_hbm, nidx_hbm, zn_hbm, outn_hbm,
             ndata, nidx_b, acc_n, sem0, sem1):
    cid = lax.axis_index("c")
    sid = lax.axis_index("s")
    w = sid * 2 + cid
    sems = (sem0, sem1)

    @pl.when(sid == 0)
    def _():
        pltpu.sync_copy(zn_hbm, acc_n)

    plsc.subcore_barrier()

    def n_start(i, buf):
        base = pl.multiple_of(w * N_PER_W + i * NB, 128)
        pltpu.async_copy(x_hbm.at[pl.ds(base, NB)], ndata.at[buf], sems[buf])
        pltpu.async_copy(nidx_hbm.at[pl.ds(base, NB)], nidx_b.at[buf],
                         sems[buf])

    def n_wait(buf):
        pltpu.make_async_copy(x_hbm.at[pl.ds(0, NB)], ndata.at[buf],
                              sems[buf]).wait()
        pltpu.make_async_copy(nidx_hbm.at[pl.ds(0, NB)], nidx_b.at[buf],
                              sems[buf]).wait()

    def n_scat(buf):
        pltpu.sync_copy(ndata.at[buf], acc_n.at[nidx_b.at[buf]], add=True)

    n_start(0, 0)

    def n_pair(p, carry):
        n_start(2 * p + 1, 1)
        n_wait(0)
        n_scat(0)

        @pl.when(p < N_BLOCKS // 2 - 1)
        def _():
            n_start(2 * p + 2, 0)

        n_wait(1)
        n_scat(1)
        return carry

    lax.fori_loop(0, N_BLOCKS // 2, n_pair, 0)

    @pl.when(w == 0)
    def _():
        def body(k, carry):
            base = pl.multiple_of(N_EXTRA_BASE + k * NB, 128)
            pltpu.sync_copy(x_hbm.at[pl.ds(base, NB)], ndata.at[0])
            pltpu.sync_copy(nidx_hbm.at[pl.ds(base, NB)], nidx_b.at[0])
            n_scat(0)
            return carry

        lax.fori_loop(0, 4, body, 0)

    @pl.when(w == 1)
    def _():
        for base in (99328, 99584):
            pltpu.sync_copy(x_hbm.at[pl.ds(base, NB)], ndata.at[0])
            pltpu.sync_copy(nidx_hbm.at[pl.ds(base, NB)], nidx_b.at[0])
            n_scat(0)
        pltpu.sync_copy(x_hbm.at[pl.ds(N_TAIL_BASE, N_TAIL_ROWS)],
                        ndata.at[0, pl.ds(0, N_TAIL_ROWS)])
        pltpu.sync_copy(nidx_hbm.at[pl.ds(N_TAIL_BASE, NB)], nidx_b.at[0])
        n_scat(0)

    plsc.subcore_barrier()

    @pl.when(sid == 0)
    def _():
        pltpu.sync_copy(acc_n.at[pl.ds(0, N_GRAPHS)], outn_hbm.at[cid])


_sc_node_sums = functools.partial(
    pl.kernel,
    out_type=jax.ShapeDtypeStruct((2, N_GRAPHS, D_NODE), jnp.float32),
    mesh=plsc.VectorSubcoreMesh(core_axis_name="c", subcore_axis_name="s",
                                num_cores=2, num_subcores=16),
    compiler_params=pltpu.CompilerParams(use_tc_tiling_on_sc=False),
    scratch_types=[
        pltpu.VMEM((2, NB, D_NODE), jnp.float32),
        pltpu.VMEM((2, NB), jnp.int32),
        pltpu.VMEM_SHARED((NSEG, D_NODE), jnp.float32),
        pltpu.SemaphoreType.DMA,
        pltpu.SemaphoreType.DMA,
    ],
)(_sc_body)


ESUB = 4000


def _edge_body(et_ref, idx_ref, out_ref):
    i = pl.program_id(0)

    @pl.when(i == 0)
    def _():
        out_ref[...] = jnp.zeros_like(out_ref)

    ids = idx_ref[0].astype(jnp.int16)
    segs = jax.lax.broadcasted_iota(jnp.int16, (N_GRAPHS, ESUB), 0)
    etb = et_ref[...].astype(jnp.bfloat16)
    acc = jnp.zeros((N_GRAPHS, D_EDGE), jnp.float32)
    for r in range(EBLK // ESUB):
        onehot = jnp.where(ids[r:r + 1, :] == segs, jnp.bfloat16(1),
                           jnp.bfloat16(0))
        acc += jax.lax.dot_general(
            onehot, etb[:, r * ESUB:(r + 1) * ESUB],
            (((1,), (1,)), ((), ())), preferred_element_type=jnp.float32)
    out_ref[...] += acc


def _mlp_body(u_ref, pn_ref, pe_ref, w1_ref, b1_ref, w2_ref, b2_ref,
              g_ref, bt_ref, out_ref):
    nodes = pn_ref[0] + pn_ref[1]
    h = (jnp.dot(u_ref[...], w1_ref[0:32, :],
                 preferred_element_type=jnp.float32)
         + jnp.dot(nodes, w1_ref[32:160, :],
                   preferred_element_type=jnp.float32)
         + jnp.dot(pe_ref[...], w1_ref[160:176, :],
                   preferred_element_type=jnp.float32)
         + b1_ref[...])
    h = jnp.maximum(h, 0.0)
    h = jnp.dot(h, w2_ref[...], preferred_element_type=jnp.float32) + b2_ref[...]
    mean = jnp.mean(h, axis=-1, keepdims=True)
    var = jnp.mean((h - mean) ** 2, axis=-1, keepdims=True)
    out_ref[...] = ((h - mean) * jax.lax.rsqrt(var + 1e-5) * g_ref[...]
                    + bt_ref[...])


def kernel(x, edge_index, edge_attr, u, node_index, W1, b1, W2, b2,
           ln_gamma, ln_beta):
    nidx_pad = jnp.concatenate(
        [node_index, jnp.full((N_PAD - N_NODES,), TRASH, dtype=jnp.int32)])
    zn = jnp.zeros((NSEG, D_NODE), jnp.float32)

    pn = _sc_node_sums(x, nidx_pad, zn)

    et = edge_attr.T
    eidx3 = edge_index.reshape(E_GRID, EBLK // ESUB, ESUB)
    eagg = pl.pallas_call(
        _edge_body,
        grid=(E_GRID,),
        in_specs=[
            pl.BlockSpec((D_EDGE, EBLK), lambda i: (0, i)),
            pl.BlockSpec((1, EBLK // ESUB, ESUB), lambda i: (i, 0, 0)),
        ],
        out_specs=pl.BlockSpec((N_GRAPHS, D_EDGE), lambda i: (0, 0)),
        out_shape=jax.ShapeDtypeStruct((N_GRAPHS, D_EDGE), jnp.float32),
    )(et, eidx3)

    out = pl.pallas_call(
        _mlp_body,
        out_shape=jax.ShapeDtypeStruct((N_GRAPHS, 128), jnp.float32),
    )(u, pn, eagg, W1, b1.reshape(1, -1), W2, b2.reshape(1, -1),
      ln_gamma.reshape(1, -1), ln_beta.reshape(1, -1))
    return out


# speedup vs baseline: 2.0811x; 1.0597x over previous
import functools

import jax
import jax.numpy as jnp
from jax import lax
from jax.experimental import pallas as pl
from jax.experimental.pallas import tpu as pltpu
from jax.experimental.pallas import tpu_sc as plsc

N_NODES = 100000
N_EDGES = 1600000
D_NODE = 128
D_EDGE = 16
N_GRAPHS = 64
NSEG = N_GRAPHS + 1
TRASH = N_GRAPHS

NW = 32

NB = 256
N_BLOCKS = 12
N_PER_W = N_BLOCKS * NB
N_EXTRA_BASE = NW * N_PER_W
N_TAIL_BASE = 99840
N_TAIL_ROWS = N_NODES - N_TAIL_BASE
N_PAD = 100096

EBLK = 160000
E_GRID = N_EDGES // EBLK


def _sc_body(x_hbm, nidx_hbm, zn_hbm, outn_hbm,
             ndata, nidx_b, acc_n, sem0, sem1):
    cid = lax.axis_index("c")
    sid = lax.axis_index("s")
    w = sid * 2 + cid
    sems = (sem0, sem1)

    @pl.when(sid == 0)
    def _():
        pltpu.sync_copy(zn_hbm, acc_n)

    plsc.subcore_barrier()

    def n_start(i, buf):
        base = pl.multiple_of(w * N_PER_W + i * NB, 128)
        pltpu.async_copy(x_hbm.at[pl.ds(base, NB)], ndata.at[buf], sems[buf])
        pltpu.async_copy(nidx_hbm.at[pl.ds(base, NB)], nidx_b.at[buf],
                         sems[buf])

    def n_wait(buf):
        pltpu.make_async_copy(x_hbm.at[pl.ds(0, NB)], ndata.at[buf],
                              sems[buf]).wait()
        pltpu.make_async_copy(nidx_hbm.at[pl.ds(0, NB)], nidx_b.at[buf],
                              sems[buf]).wait()

    def n_scat(buf):
        pltpu.sync_copy(ndata.at[buf], acc_n.at[nidx_b.at[buf]], add=True)

    n_start(0, 0)

    def n_pair(p, carry):
        n_start(2 * p + 1, 1)
        n_wait(0)
        n_scat(0)

        @pl.when(p < N_BLOCKS // 2 - 1)
        def _():
            n_start(2 * p + 2, 0)

        n_wait(1)
        n_scat(1)
        return carry

    lax.fori_loop(0, N_BLOCKS // 2, n_pair, 0)

    @pl.when(w == 0)
    def _():
        def body(k, carry):
            base = pl.multiple_of(N_EXTRA_BASE + k * NB, 128)
            pltpu.sync_copy(x_hbm.at[pl.ds(base, NB)], ndata.at[0])
            pltpu.sync_copy(nidx_hbm.at[pl.ds(base, NB)], nidx_b.at[0])
            n_scat(0)
            return carry

        lax.fori_loop(0, 4, body, 0)

    @pl.when(w == 1)
    def _():
        for base in (99328, 99584):
            pltpu.sync_copy(x_hbm.at[pl.ds(base, NB)], ndata.at[0])
            pltpu.sync_copy(nidx_hbm.at[pl.ds(base, NB)], nidx_b.at[0])
            n_scat(0)
        pltpu.sync_copy(x_hbm.at[pl.ds(N_TAIL_BASE, N_TAIL_ROWS)],
                        ndata.at[0, pl.ds(0, N_TAIL_ROWS)])
        pltpu.sync_copy(nidx_hbm.at[pl.ds(N_TAIL_BASE, NB)], nidx_b.at[0])
        n_scat(0)

    plsc.subcore_barrier()

    @pl.when(sid == 0)
    def _():
        pltpu.sync_copy(acc_n.at[pl.ds(0, N_GRAPHS)], outn_hbm.at[cid])


_sc_node_sums = functools.partial(
    pl.kernel,
    out_type=jax.ShapeDtypeStruct((2, N_GRAPHS, D_NODE), jnp.float32),
    mesh=plsc.VectorSubcoreMesh(core_axis_name="c", subcore_axis_name="s",
                                num_cores=2, num_subcores=16),
    compiler_params=pltpu.CompilerParams(use_tc_tiling_on_sc=False),
    scratch_types=[
        pltpu.VMEM((2, NB, D_NODE), jnp.float32),
        pltpu.VMEM((2, NB), jnp.int32),
        pltpu.VMEM_SHARED((NSEG, D_NODE), jnp.float32),
        pltpu.SemaphoreType.DMA,
        pltpu.SemaphoreType.DMA,
    ],
)(_sc_body)


ESUB = 20000


def _edge_body(et_ref, idx_ref, out_ref):
    i = pl.program_id(0)

    @pl.when(i == 0)
    def _():
        out_ref[...] = jnp.zeros_like(out_ref)

    ids = idx_ref[0].astype(jnp.int16)
    segs = jax.lax.broadcasted_iota(jnp.int16, (N_GRAPHS, ESUB), 0)
    etb = et_ref[...].astype(jnp.bfloat16)
    acc = jnp.zeros((N_GRAPHS, D_EDGE), jnp.float32)
    for r in range(EBLK // ESUB):
        onehot = jnp.where(ids[r:r + 1, :] == segs, jnp.bfloat16(1),
                           jnp.bfloat16(0))
        acc += jax.lax.dot_general(
            onehot, etb[:, r * ESUB:(r + 1) * ESUB],
            (((1,), (1,)), ((), ())), preferred_element_type=jnp.float32)
    out_ref[...] += acc


def _mlp_body(u_ref, pn_ref, pe_ref, w1_ref, b1_ref, w2_ref, b2_ref,
              g_ref, bt_ref, out_ref):
    nodes = pn_ref[0] + pn_ref[1]
    h = (jnp.dot(u_ref[...], w1_ref[0:32, :],
                 preferred_element_type=jnp.float32)
         + jnp.dot(nodes, w1_ref[32:160, :],
                   preferred_element_type=jnp.float32)
         + jnp.dot(pe_ref[...], w1_ref[160:176, :],
                   preferred_element_type=jnp.float32)
         + b1_ref[...])
    h = jnp.maximum(h, 0.0)
    h = jnp.dot(h, w2_ref[...], preferred_element_type=jnp.float32) + b2_ref[...]
    mean = jnp.mean(h, axis=-1, keepdims=True)
    var = jnp.mean((h - mean) ** 2, axis=-1, keepdims=True)
    out_ref[...] = ((h - mean) * jax.lax.rsqrt(var + 1e-5) * g_ref[...]
                    + bt_ref[...])


def kernel(x, edge_index, edge_attr, u, node_index, W1, b1, W2, b2,
           ln_gamma, ln_beta):
    nidx_pad = jnp.concatenate(
        [node_index, jnp.full((N_PAD - N_NODES,), TRASH, dtype=jnp.int32)])
    zn = jnp.zeros((NSEG, D_NODE), jnp.float32)

    pn = _sc_node_sums(x, nidx_pad, zn)

    et = edge_attr.T
    eidx3 = edge_index.reshape(E_GRID, EBLK // ESUB, ESUB)
    eagg = pl.pallas_call(
        _edge_body,
        grid=(E_GRID,),
        in_specs=[
            pl.BlockSpec((D_EDGE, EBLK), lambda i: (0, i)),
            pl.BlockSpec((1, EBLK // ESUB, ESUB), lambda i: (i, 0, 0)),
        ],
        out_specs=pl.BlockSpec((N_GRAPHS, D_EDGE), lambda i: (0, 0)),
        out_shape=jax.ShapeDtypeStruct((N_GRAPHS, D_EDGE), jnp.float32),
    )(et, eidx3)

    out = pl.pallas_call(
        _mlp_body,
        out_shape=jax.ShapeDtypeStruct((N_GRAPHS, 128), jnp.float32),
    )(u, pn, eagg, W1, b1.reshape(1, -1), W2, b2.reshape(1, -1),
      ln_gamma.reshape(1, -1), ln_beta.reshape(1, -1))
    return out
